# baseline (speedup 1.0000x reference)
import functools

import numpy as np

import jax
import jax.numpy as jnp
from jax import lax
from jax.experimental import pallas as pl
from jax.experimental.pallas import tpu as pltpu
from jax.experimental.pallas import tpu_sc as plsc

HIDDEN = 128
VOCAB = 100000
NPOS = 513
PAD = 0
EPS = 1e-5

NC = 2
NS = 16
NW = NC * NS
L = 16
NBLK = HIDDEN // L

C = 128

_RSQRT_MAGIC = np.int32(0x5F3759DF)


def _rsqrt(a):
    ai = lax.bitcast_convert_type(a, jnp.int32)
    y = lax.bitcast_convert_type(_RSQRT_MAGIC - (ai >> 1), jnp.float32)
    ha = a * 0.5
    for _ in range(3):
        y = y * (1.5 - ha * y * y)
    return y


def _make_kernel(n_tokens):
    assert n_tokens % (NW * C) == 0
    per_w = n_tokens // NW
    n_chunks = per_w // C
    assert n_chunks % 2 == 0 and n_chunks >= 4

    mesh = plsc.VectorSubcoreMesh(
        core_axis_name="c", subcore_axis_name="s",
        num_cores=NC, num_subcores=NS,
    )

    @functools.partial(
        pl.kernel,
        out_type=(
            jax.ShapeDtypeStruct((n_tokens, HIDDEN), jnp.float32),
            jax.ShapeDtypeStruct((n_tokens,), jnp.int32),
        ),
        mesh=mesh,
        compiler_params=pltpu.CompilerParams(needs_layout_passes=False),
        scratch_types=[
            pltpu.VMEM((NPOS, HIDDEN), jnp.float32),
            pltpu.VMEM((2, C), jnp.int32),
            pltpu.VMEM((2, C), jnp.int32),
            pltpu.VMEM((2, C, HIDDEN), jnp.float32),
            pltpu.VMEM((C,), jnp.int32),
            pltpu.VMEM((HIDDEN,), jnp.float32),
            pltpu.VMEM((HIDDEN,), jnp.float32),
            pltpu.SemaphoreType.DMA,
            pltpu.SemaphoreType.DMA,
        ],
    )
    def emb_kernel(idw_hbm, idp_hbm, wtab_hbm, ptab_hbm, g_hbm, b_hbm,
                   out_hbm, mask_hbm,
                   ptab_v, idw_v, idp_v, wrows, mvec, gv, bv,
                   sem0, sem1):
        wid = lax.axis_index("s") * NC + lax.axis_index("c")
        base = wid * per_w

        pltpu.sync_copy(ptab_hbm, ptab_v)
        pltpu.sync_copy(g_hbm, gv)
        pltpu.sync_copy(b_hbm, bv)
        gs = [gv[pl.ds(L * e, L)] for e in range(NBLK)]
        bs = [bv[pl.ds(L * e, L)] for e in range(NBLK)]
        cols = [lax.iota(jnp.int32, L) + L * e for e in range(NBLK)]
        sems = (sem0, sem1)

        def copy_ids(ci, slot):
            off = base + ci * C
            pltpu.sync_copy(idw_hbm.at[pl.ds(off, C)], idw_v.at[slot])
            pltpu.sync_copy(idp_hbm.at[pl.ds(off, C)], idp_v.at[slot])

        def issue_gather(slot):
            return pltpu.async_copy(wtab_hbm.at[idw_v.at[slot]],
                                    wrows.at[slot], sems[slot])

        def compute_store(ci, slot):
            off = base + ci * C
            idps = idp_v.at[slot]
            idws = idw_v.at[slot]
            rows = wrows.at[slot]

            def mask_body(g, c2):
                v = idws[pl.ds(g * L, L)]
                mvec[pl.ds(g * L, L)] = 1 - jnp.minimum(v, 1)
                return c2

            lax.fori_loop(0, C // L, mask_body, 0)

            def tok_body(t, c2):
                tidx = jnp.full((L,), t, jnp.int32)
                ip = plsc.load_gather(idps, [tidx])
                xs = []
                for e in range(NBLK):
                    wv = rows[t, pl.ds(L * e, L)]
                    pv = plsc.load_gather(ptab_v, [ip, cols[e]])
                    xs.append(wv + pv)
                s = xs[0]
                ssq = xs[0] * xs[0]
                for e in range(1, NBLK):
                    s = s + xs[e]
                    ssq = ssq + xs[e] * xs[e]
                mean = jnp.sum(s) * (1.0 / HIDDEN)
                var = jnp.sum(ssq) * (1.0 / HIDDEN) - mean * mean
                inv = _rsqrt(var + EPS)
                for e in range(NBLK):
                    t1 = gs[e] * inv
                    rows[t, pl.ds(L * e, L)] = (xs[e] - mean) * t1 + bs[e]
                return c2

            lax.fori_loop(0, C, tok_body, 0)

            pltpu.sync_copy(rows, out_hbm.at[pl.ds(off, C)])
            pltpu.sync_copy(mvec, mask_hbm.at[pl.ds(off, C)])

        copy_ids(0, 0)
        copy_ids(1, 1)
        cw0 = issue_gather(0)

        def pair_body(k, carry):
            i0 = 2 * k
            issue_gather(1)
            pltpu.make_async_copy(wtab_hbm.at[idw_v.at[0]],
                                  wrows.at[0], sem0).wait()
            compute_store(i0, 0)
            copy_ids(i0 + 2, 0)
            issue_gather(0)
            pltpu.make_async_copy(wtab_hbm.at[idw_v.at[1]],
                                  wrows.at[1], sem1).wait()
            compute_store(i0 + 1, 1)
            copy_ids(i0 + 3, 1)
            return carry

        lax.fori_loop(0, n_chunks // 2 - 1, pair_body, 0)

        issue_gather(1)
        pltpu.make_async_copy(wtab_hbm.at[idw_v.at[0]],
                              wrows.at[0], sem0).wait()
        compute_store(n_chunks - 2, 0)
        pltpu.make_async_copy(wtab_hbm.at[idw_v.at[1]],
                              wrows.at[1], sem1).wait()
        compute_store(n_chunks - 1, 1)

    return emb_kernel


@jax.jit
def _run(idw, idp, word_emb, pos_emb, ln_gamma, ln_beta):
    n_tokens = idw.shape[0]
    w = word_emb.at[PAD].set(0.0)
    p = pos_emb.at[PAD].set(0.0)
    return _make_kernel(n_tokens)(idw, idp, w, p, ln_gamma, ln_beta)


def kernel(uttr_ids_list, position_ids_list, word_emb, pos_emb, ln_gamma,
           ln_beta):
    B, S = uttr_ids_list.shape
    n = B * S
    out, mask = _run(uttr_ids_list.reshape(n), position_ids_list.reshape(n),
                     word_emb, pos_emb, ln_gamma, ln_beta)
    return out.reshape(B, S, HIDDEN), mask.reshape(B, S).astype(bool)


# speedup vs baseline: 1.1372x; 1.1372x over previous
import functools

import numpy as np

import jax
import jax.numpy as jnp
from jax import lax
from jax.experimental import pallas as pl
from jax.experimental.pallas import tpu as pltpu
from jax.experimental.pallas import tpu_sc as plsc

HIDDEN = 128
PAD = 0
EPS = 1e-5

NC = 2
NS = 16
NW = NC * NS
L = 16
NBLK = HIDDEN // L

C = 128

_RSQRT_MAGIC = np.int32(0x5F3759DF)


def _rsqrt(a):
    ai = lax.bitcast_convert_type(a, jnp.int32)
    y = lax.bitcast_convert_type(_RSQRT_MAGIC - (ai >> 1), jnp.float32)
    ha = a * 0.5
    for _ in range(3):
        y = y * (1.5 - ha * y * y)
    return y


def _make_kernel(n_tokens):
    assert n_tokens % (NW * C) == 0
    per_w = n_tokens // NW
    n_chunks = per_w // C
    assert n_chunks % 2 == 0 and n_chunks >= 4

    mesh = plsc.VectorSubcoreMesh(
        core_axis_name="c", subcore_axis_name="s",
        num_cores=NC, num_subcores=NS,
    )

    @functools.partial(
        pl.kernel,
        out_type=(
            jax.ShapeDtypeStruct((n_tokens, HIDDEN), jnp.float32),
            jax.ShapeDtypeStruct((n_tokens,), jnp.int32),
        ),
        mesh=mesh,
        compiler_params=pltpu.CompilerParams(needs_layout_passes=False),
        scratch_types=[
            pltpu.VMEM((2, C), jnp.int32),
            pltpu.VMEM((2, C), jnp.int32),
            pltpu.VMEM((2, C, HIDDEN), jnp.float32),
            pltpu.VMEM((2, C, HIDDEN), jnp.float32),
            pltpu.VMEM((C,), jnp.int32),
            pltpu.VMEM((HIDDEN,), jnp.float32),
            pltpu.VMEM((HIDDEN,), jnp.float32),
            pltpu.SemaphoreType.DMA,
            pltpu.SemaphoreType.DMA,
        ],
    )
    def emb_kernel(idw_hbm, idp_hbm, wtab_hbm, ptab_hbm, g_hbm, b_hbm,
                   out_hbm, mask_hbm,
                   idw_v, idp_v, wrows, prows, mvec, gv, bv,
                   sem0, sem1):
        wid = lax.axis_index("s") * NC + lax.axis_index("c")
        base = wid * per_w

        pltpu.sync_copy(g_hbm, gv)
        pltpu.sync_copy(b_hbm, bv)
        gs = [gv[pl.ds(L * e, L)] for e in range(NBLK)]
        bs = [bv[pl.ds(L * e, L)] for e in range(NBLK)]
        sems = (sem0, sem1)

        def copy_ids(ci, slot):
            off = base + ci * C
            pltpu.sync_copy(idw_hbm.at[pl.ds(off, C)], idw_v.at[slot])
            pltpu.sync_copy(idp_hbm.at[pl.ds(off, C)], idp_v.at[slot])

        def issue_gathers(slot):
            pltpu.async_copy(wtab_hbm.at[idw_v.at[slot]],
                             wrows.at[slot], sems[slot])
            pltpu.async_copy(ptab_hbm.at[idp_v.at[slot]],
                             prows.at[slot], sems[slot])

        def wait_gathers(slot):
            pltpu.make_async_copy(wtab_hbm.at[idw_v.at[slot]],
                                  wrows.at[slot], sems[slot]).wait()
            pltpu.make_async_copy(ptab_hbm.at[idp_v.at[slot]],
                                  prows.at[slot], sems[slot]).wait()

        def compute_store(ci, slot):
            off = base + ci * C
            idws = idw_v.at[slot]
            rows = wrows.at[slot]
            rowsp = prows.at[slot]

            def mask_body(g, c2):
                v = idws[pl.ds(g * L, L)]
                mvec[pl.ds(g * L, L)] = 1 - jnp.minimum(v, 1)
                return c2

            lax.fori_loop(0, C // L, mask_body, 0)

            def tok_body(t, c2):
                xs = []
                for e in range(NBLK):
                    xs.append(rows[t, pl.ds(L * e, L)]
                              + rowsp[t, pl.ds(L * e, L)])
                s = xs[0]
                ssq = xs[0] * xs[0]
                for e in range(1, NBLK):
                    s = s + xs[e]
                    ssq = ssq + xs[e] * xs[e]
                mean = jnp.sum(s) * (1.0 / HIDDEN)
                var = jnp.sum(ssq) * (1.0 / HIDDEN) - mean * mean
                inv = _rsqrt(var + EPS)
                for e in range(NBLK):
                    t1 = gs[e] * inv
                    rows[t, pl.ds(L * e, L)] = (xs[e] - mean) * t1 + bs[e]
                return c2

            lax.fori_loop(0, C, tok_body, 0)

            pltpu.sync_copy(rows, out_hbm.at[pl.ds(off, C)])
            pltpu.sync_copy(mvec, mask_hbm.at[pl.ds(off, C)])

        copy_ids(0, 0)
        copy_ids(1, 1)
        issue_gathers(0)

        def pair_body(k, carry):
            i0 = 2 * k
            issue_gathers(1)
            wait_gathers(0)
            compute_store(i0, 0)
            copy_ids(i0 + 2, 0)
            issue_gathers(0)
            wait_gathers(1)
            compute_store(i0 + 1, 1)
            copy_ids(i0 + 3, 1)
            return carry

        lax.fori_loop(0, n_chunks // 2 - 1, pair_body, 0)

        issue_gathers(1)
        wait_gathers(0)
        compute_store(n_chunks - 2, 0)
        wait_gathers(1)
        compute_store(n_chunks - 1, 1)

    return emb_kernel


@jax.jit
def _run(idw, idp, word_emb, pos_emb, ln_gamma, ln_beta):
    n_tokens = idw.shape[0]
    w = word_emb.at[PAD].set(0.0)
    p = pos_emb.at[PAD].set(0.0)
    return _make_kernel(n_tokens)(idw, idp, w, p, ln_gamma, ln_beta)


def kernel(uttr_ids_list, position_ids_list, word_emb, pos_emb, ln_gamma,
           ln_beta):
    B, S = uttr_ids_list.shape
    n = B * S
    out, mask = _run(uttr_ids_list.reshape(n), position_ids_list.reshape(n),
                     word_emb, pos_emb, ln_gamma, ln_beta)
    return out.reshape(B, S, HIDDEN), mask.reshape(B, S).astype(bool)


# speedup vs baseline: 1.4738x; 1.2960x over previous
import functools

import numpy as np

import jax
import jax.numpy as jnp
from jax import lax
from jax.experimental import pallas as pl
from jax.experimental.pallas import tpu as pltpu
from jax.experimental.pallas import tpu_sc as plsc

HIDDEN = 128
PAD = 0
EPS = 1e-5

NC = 2
NS = 16
NW = NC * NS
L = 16
NBLK = HIDDEN // L

C = 128

_RSQRT_MAGIC = np.int32(0x5F3759DF)


def _rsqrt(a):
    ai = lax.bitcast_convert_type(a, jnp.int32)
    y = lax.bitcast_convert_type(_RSQRT_MAGIC - (ai >> 1), jnp.float32)
    ha = a * 0.5
    for _ in range(3):
        y = y * (1.5 - ha * y * y)
    return y


def _make_kernel(n_tokens):
    assert n_tokens % (NW * C) == 0
    per_w = n_tokens // NW
    n_chunks = per_w // C
    assert n_chunks % 2 == 0 and n_chunks >= 4

    mesh = plsc.VectorSubcoreMesh(
        core_axis_name="c", subcore_axis_name="s",
        num_cores=NC, num_subcores=NS,
    )

    @functools.partial(
        pl.kernel,
        out_type=(
            jax.ShapeDtypeStruct((n_tokens, HIDDEN), jnp.float32),
            jax.ShapeDtypeStruct((n_tokens,), jnp.int32),
        ),
        mesh=mesh,
        compiler_params=pltpu.CompilerParams(needs_layout_passes=False),
        scratch_types=[
            pltpu.VMEM((2, C), jnp.int32),
            pltpu.VMEM((2, C), jnp.int32),
            pltpu.VMEM((2, C, HIDDEN), jnp.float32),
            pltpu.VMEM((2, C, HIDDEN), jnp.float32),
            pltpu.VMEM((C, HIDDEN), jnp.float32),
            pltpu.VMEM((C,), jnp.int32),
            pltpu.VMEM((HIDDEN,), jnp.float32),
            pltpu.VMEM((HIDDEN,), jnp.float32),
            pltpu.SemaphoreType.DMA,
            pltpu.SemaphoreType.DMA,
        ],
    )
    def emb_kernel(idw_hbm, idp_hbm, wtab_hbm, ptab_hbm, g_hbm, b_hbm,
                   out_hbm, mask_hbm,
                   idw_v, idp_v, wrows, prows, orows, mvec, gv, bv,
                   sem0, sem1):
        wid = lax.axis_index("s") * NC + lax.axis_index("c")
        base = wid * per_w

        pltpu.sync_copy(g_hbm, gv)
        pltpu.sync_copy(b_hbm, bv)
        gs = [gv[pl.ds(L * e, L)] for e in range(NBLK)]
        bs = [bv[pl.ds(L * e, L)] for e in range(NBLK)]
        sems = (sem0, sem1)

        def copy_ids(ci, slot):
            off = base + ci * C
            pltpu.sync_copy(idw_hbm.at[pl.ds(off, C)], idw_v.at[slot])
            pltpu.sync_copy(idp_hbm.at[pl.ds(off, C)], idp_v.at[slot])

        def issue_gathers(slot):
            pltpu.async_copy(wtab_hbm.at[idw_v.at[slot]],
                             wrows.at[slot], sems[slot])
            pltpu.async_copy(ptab_hbm.at[idp_v.at[slot]],
                             prows.at[slot], sems[slot])

        def wait_gathers(slot):
            pltpu.make_async_copy(wtab_hbm.at[idw_v.at[slot]],
                                  wrows.at[slot], sems[slot]).wait()
            pltpu.make_async_copy(ptab_hbm.at[idp_v.at[slot]],
                                  prows.at[slot], sems[slot]).wait()

        def compute_store(ci, slot):
            off = base + ci * C
            idws = idw_v.at[slot]
            rows = wrows.at[slot]
            rowsp = prows.at[slot]

            def mask_body(g, c2):
                v = idws[pl.ds(g * L, L)]
                mvec[pl.ds(g * L, L)] = 1 - jnp.minimum(v, 1)
                return c2

            lax.fori_loop(0, C // L, mask_body, 0)

            def tok_body(t, c2):
                xs = []
                for e in range(NBLK):
                    xs.append(rows[t, pl.ds(L * e, L)]
                              + rowsp[t, pl.ds(L * e, L)])
                s = xs[0]
                ssq = xs[0] * xs[0]
                for e in range(1, NBLK):
                    s = s + xs[e]
                    ssq = ssq + xs[e] * xs[e]
                mean = jnp.sum(s) * (1.0 / HIDDEN)
                var = jnp.sum(ssq) * (1.0 / HIDDEN) - mean * mean
                inv = _rsqrt(var + EPS)
                for e in range(NBLK):
                    t1 = gs[e] * inv
                    orows[t, pl.ds(L * e, L)] = (xs[e] - mean) * t1 + bs[e]
                return c2

            lax.fori_loop(0, C, tok_body, 0, unroll=2)

            pltpu.sync_copy(orows, out_hbm.at[pl.ds(off, C)])
            pltpu.sync_copy(mvec, mask_hbm.at[pl.ds(off, C)])

        copy_ids(0, 0)
        copy_ids(1, 1)
        issue_gathers(0)

        def pair_body(k, carry):
            i0 = 2 * k
            issue_gathers(1)
            wait_gathers(0)
            compute_store(i0, 0)
            copy_ids(i0 + 2, 0)
            issue_gathers(0)
            wait_gathers(1)
            compute_store(i0 + 1, 1)
            copy_ids(i0 + 3, 1)
            return carry

        lax.fori_loop(0, n_chunks // 2 - 1, pair_body, 0)

        issue_gathers(1)
        wait_gathers(0)
        compute_store(n_chunks - 2, 0)
        wait_gathers(1)
        compute_store(n_chunks - 1, 1)

    return emb_kernel


@jax.jit
def _run(idw, idp, word_emb, pos_emb, ln_gamma, ln_beta):
    n_tokens = idw.shape[0]
    w = word_emb.at[PAD].set(0.0)
    p = pos_emb.at[PAD].set(0.0)
    return _make_kernel(n_tokens)(idw, idp, w, p, ln_gamma, ln_beta)


def kernel(uttr_ids_list, position_ids_list, word_emb, pos_emb, ln_gamma,
           ln_beta):
    B, S = uttr_ids_list.shape
    n = B * S
    out, mask = _run(uttr_ids_list.reshape(n), position_ids_list.reshape(n),
                     word_emb, pos_emb, ln_gamma, ln_beta)
    return out.reshape(B, S, HIDDEN), mask.reshape(B, S).astype(bool)
